# baseline (speedup 1.0000x reference)
import functools

import jax
import jax.numpy as jnp
from jax import lax
from jax.experimental import pallas as pl
from jax.experimental.pallas import tpu as pltpu
from jax.experimental.pallas import tpu_sc as plsc

NC = 2
NS = 16
K = 128


def _mesh():
    return plsc.VectorSubcoreMesh(core_axis_name="c", subcore_axis_name="s")


@functools.lru_cache(maxsize=None)
def _make_deg(n_pad, chunks):
    ept = n_pad // NS

    @functools.partial(
        pl.kernel,
        mesh=_mesh(),
        out_type=jax.ShapeDtypeStruct((NC * n_pad,), jnp.float32),
        scratch_types=[
            pltpu.VMEM((chunks, K), jnp.int32),
            pltpu.VMEM((K,), jnp.float32),
            pltpu.VMEM_SHARED((n_pad,), jnp.float32),
        ],
    )
    def deg_kernel(dst_hbm, ones_hbm, zeros_hbm, out_hbm, dst_v, ones_v, deg_sh):
        c = lax.axis_index("c")
        s = lax.axis_index("s")
        wid = c * NS + s

        @pl.when(s == 0)
        def _():
            pltpu.sync_copy(zeros_hbm, deg_sh)

        pltpu.sync_copy(dst_hbm.at[wid], dst_v)
        pltpu.sync_copy(ones_hbm, ones_v)
        plsc.subcore_barrier()

        def body(j, carry):
            pltpu.sync_copy(ones_v, deg_sh.at[dst_v.at[j]], add=True)
            return carry

        lax.fori_loop(0, chunks, body, 0)
        plsc.subcore_barrier()
        pltpu.sync_copy(deg_sh.at[pl.ds(s * ept, ept)],
                        out_hbm.at[pl.ds(c * n_pad + s * ept, ept)])

    return deg_kernel


@functools.lru_cache(maxsize=None)
def _make_prop(n_pad, d, chunks):
    ept = n_pad // NS

    @functools.partial(
        pl.kernel,
        mesh=_mesh(),
        out_type=jax.ShapeDtypeStruct((NC, n_pad, d), jnp.float32),
        scratch_types=[
            pltpu.VMEM((chunks // 2, K), jnp.int32),
            pltpu.VMEM((chunks // 2, K), jnp.int32),
            pltpu.VMEM((2, K, d), jnp.float32),
            pltpu.VMEM_SHARED((n_pad, d), jnp.float32),
            pltpu.SemaphoreType.DMA,
        ],
    )
    def prop_kernel(h_hbm, src_hbm, dst_hbm, zeros_hbm, out_hbm,
                    src_v, dst_v, rows_v, agg_sh, gsem):
        c = lax.axis_index("c")
        s = lax.axis_index("s")
        wid = c * NS + s

        pltpu.sync_copy(zeros_hbm.at[pl.ds(s * ept, ept)],
                        agg_sh.at[pl.ds(s * ept, ept)])

        hc = chunks // 2
        npairs = hc // 2
        for half in range(2):
            pltpu.sync_copy(src_hbm.at[wid, pl.ds(half * hc, hc)], src_v)
            pltpu.sync_copy(dst_hbm.at[wid, pl.ds(half * hc, hc)], dst_v)
            if half == 0:
                plsc.subcore_barrier()
            H = K // 2

            def gather(j, b):
                pltpu.async_copy(h_hbm.at[src_v.at[j, pl.ds(0, H)]],
                                 rows_v.at[b, pl.ds(0, H)], gsem)
                pltpu.async_copy(h_hbm.at[src_v.at[j, pl.ds(H, H)]],
                                 rows_v.at[b, pl.ds(H, H)], gsem)

            def wait_gather(b):
                pltpu.make_async_copy(h_hbm.at[src_v.at[0, pl.ds(0, H)]],
                                      rows_v.at[b], gsem).wait()

            gather(0, 0)

            def body(jj, carry):
                j0 = 2 * jj
                gather(j0 + 1, 1)
                wait_gather(0)
                pltpu.sync_copy(rows_v.at[0], agg_sh.at[dst_v.at[j0]], add=True)

                @pl.when(jj + 1 < npairs)
                def _():
                    gather(j0 + 2, 0)

                wait_gather(1)
                pltpu.sync_copy(rows_v.at[1], agg_sh.at[dst_v.at[j0 + 1]],
                                add=True)
                return carry

            lax.fori_loop(0, npairs, body, 0)
        plsc.subcore_barrier()
        pltpu.sync_copy(agg_sh.at[pl.ds(s * ept, ept)],
                        out_hbm.at[c, pl.ds(s * ept, ept)])

    return prop_kernel


def _dinv(d0_ref, d1_ref, n):
    d = d0_ref[...] + d1_ref[...] + 1.0
    return lax.rsqrt(d)[:n]


@functools.lru_cache(maxsize=None)
def _make_tc_in(n, d_in, d_hid):
    def body(x_ref, w_ref, d0_ref, d1_ref, out_ref):
        dinv = _dinv(d0_ref, d1_ref, n)
        h = jnp.dot(x_ref[...], w_ref[...], preferred_element_type=jnp.float32)
        out_ref[...] = h * dinv

    return pl.pallas_call(
        body, out_shape=jax.ShapeDtypeStruct((n, d_hid), jnp.float32))


@functools.lru_cache(maxsize=None)
def _make_tc_mid(n, n_pad, d, d_next):
    def body(p_ref, hp_ref, d0_ref, d1_ref, b_ref, g_ref, be_ref, w_ref, out_ref):
        dinv = _dinv(d0_ref, d1_ref, n)
        a = (p_ref[0, :n] + p_ref[1, :n] + hp_ref[...]) * dinv + b_ref[...]
        mu = jnp.mean(a, axis=0, keepdims=True)
        var = jnp.mean((a - mu) * (a - mu), axis=0, keepdims=True)
        y = (a - mu) * lax.rsqrt(var + 1e-5) * g_ref[...] + be_ref[...]
        y = jnp.maximum(y, 0.0)
        h = jnp.dot(y, w_ref[...], preferred_element_type=jnp.float32)
        out_ref[...] = h * dinv

    return pl.pallas_call(
        body, out_shape=jax.ShapeDtypeStruct((n, d_next), jnp.float32))


@functools.lru_cache(maxsize=None)
def _make_tc_mid_now(n, n_pad, d):
    def body(p_ref, hp_ref, d0_ref, d1_ref, b_ref, g_ref, be_ref, out_ref):
        dinv = _dinv(d0_ref, d1_ref, n)
        a = (p_ref[0, :n] + p_ref[1, :n] + hp_ref[...]) * dinv + b_ref[...]
        mu = jnp.mean(a, axis=0, keepdims=True)
        var = jnp.mean((a - mu) * (a - mu), axis=0, keepdims=True)
        y = (a - mu) * lax.rsqrt(var + 1e-5) * g_ref[...] + be_ref[...]
        y = jnp.maximum(y, 0.0)
        out_ref[...] = y * dinv

    return pl.pallas_call(
        body, out_shape=jax.ShapeDtypeStruct((n, d), jnp.float32))


@functools.lru_cache(maxsize=None)
def _make_tc_out(n, n_pad, d, d_out):
    def body(p_ref, hp_ref, d0_ref, d1_ref, w_ref, b_ref, out_ref):
        dinv = _dinv(d0_ref, d1_ref, n)
        agg = (p_ref[0, :n] + p_ref[1, :n] + hp_ref[...]) * dinv
        a = jnp.dot(agg, w_ref[...], preferred_element_type=jnp.float32)
        a = a + b_ref[...]
        m = jnp.max(a, axis=-1, keepdims=True)
        e = jnp.exp(a - m)
        lse = jnp.log(jnp.sum(e, axis=-1, keepdims=True))
        out_ref[...] = a - m - lse

    return pl.pallas_call(
        body, out_shape=jax.ShapeDtypeStruct((n, d_out), jnp.float32))


def kernel(x, edge_index, W1, b1, g1, be1, W2, b2, g2, be2, W3, b3):
    n, d_in = x.shape
    e = edge_index.shape[1]
    d_hid = W1.shape[1]
    d_out = W3.shape[1]

    n_pad = ((n + 256) // 256) * 256
    epw = NC * NS * K * 4
    e_pad = ((e + epw - 1) // epw) * epw
    chunks = e_pad // (NC * NS * K)
    pad = e_pad - e

    pad_idx = jnp.arange(pad, dtype=jnp.int32)
    src = jnp.concatenate([edge_index[0], pad_idx % n])
    dst = jnp.concatenate([edge_index[1], n + pad_idx % (n_pad - n)])
    src3 = src.reshape(NC * NS, chunks, K)
    dst3 = dst.reshape(NC * NS, chunks, K)

    zeros1 = jnp.zeros((n_pad,), jnp.float32)
    ones_k = jnp.ones((K,), jnp.float32)
    zeros_h = jnp.zeros((n_pad, d_hid), jnp.float32)

    deg_flat = _make_deg(n_pad, chunks)(dst3, ones_k, zeros1)
    deg0 = deg_flat[:n_pad].reshape(n_pad, 1)
    deg1 = deg_flat[n_pad:].reshape(n_pad, 1)

    h1p = _make_tc_in(n, d_in, d_hid)(x, W1, deg0, deg1)
    p1 = _make_prop(n_pad, d_hid, chunks)(h1p, src3, dst3, zeros_h)
    h2p = _make_tc_mid(n, n_pad, d_hid, d_hid)(p1, h1p, deg0, deg1, b1, g1, be1, W2)
    p2 = _make_prop(n_pad, d_hid, chunks)(h2p, src3, dst3, zeros_h)
    y2p = _make_tc_mid_now(n, n_pad, d_hid)(p2, h2p, deg0, deg1, b2, g2, be2)
    p3 = _make_prop(n_pad, d_hid, chunks)(y2p, src3, dst3, zeros_h)
    return _make_tc_out(n, n_pad, d_hid, d_out)(p3, y2p, deg0, deg1, W3, b3)


# speedup vs baseline: 1.0015x; 1.0015x over previous
import functools

import jax
import jax.numpy as jnp
from jax import lax
from jax.experimental import pallas as pl
from jax.experimental.pallas import tpu as pltpu
from jax.experimental.pallas import tpu_sc as plsc

NC = 2
NS = 16
K = 128


def _mesh():
    return plsc.VectorSubcoreMesh(core_axis_name="c", subcore_axis_name="s")


@functools.lru_cache(maxsize=None)
def _make_deg(n_pad, chunks):
    ept = n_pad // NS

    @functools.partial(
        pl.kernel,
        mesh=_mesh(),
        out_type=jax.ShapeDtypeStruct((NC * n_pad,), jnp.float32),
        scratch_types=[
            pltpu.VMEM((chunks, K), jnp.int32),
            pltpu.VMEM((K,), jnp.float32),
            pltpu.VMEM_SHARED((n_pad,), jnp.float32),
        ],
    )
    def deg_kernel(dst_hbm, ones_hbm, zeros_hbm, out_hbm, dst_v, ones_v, deg_sh):
        c = lax.axis_index("c")
        s = lax.axis_index("s")
        wid = c * NS + s

        @pl.when(s == 0)
        def _():
            pltpu.sync_copy(zeros_hbm, deg_sh)

        pltpu.sync_copy(dst_hbm.at[wid], dst_v)
        pltpu.sync_copy(ones_hbm, ones_v)
        plsc.subcore_barrier()

        def body(j, carry):
            pltpu.sync_copy(ones_v, deg_sh.at[dst_v.at[j]], add=True)
            return carry

        lax.fori_loop(0, chunks, body, 0)
        plsc.subcore_barrier()
        pltpu.sync_copy(deg_sh.at[pl.ds(s * ept, ept)],
                        out_hbm.at[pl.ds(c * n_pad + s * ept, ept)])

    return deg_kernel


@functools.lru_cache(maxsize=None)
def _make_prop(n_pad, d, chunks):
    ept = n_pad // NS

    @functools.partial(
        pl.kernel,
        mesh=_mesh(),
        out_type=jax.ShapeDtypeStruct((NC, n_pad, d), jnp.float32),
        scratch_types=[
            pltpu.VMEM((chunks // 2, K), jnp.int32),
            pltpu.VMEM((chunks // 2, K), jnp.int32),
            pltpu.VMEM((2, K, d), jnp.float32),
            pltpu.VMEM_SHARED((n_pad, d), jnp.float32),
            pltpu.SemaphoreType.DMA,
        ],
    )
    def prop_kernel(h_hbm, src_hbm, dst_hbm, zeros_hbm, out_hbm,
                    src_v, dst_v, rows_v, agg_sh, gsem):
        c = lax.axis_index("c")
        s = lax.axis_index("s")
        wid = c * NS + s

        pltpu.sync_copy(zeros_hbm.at[pl.ds(s * ept, ept)],
                        agg_sh.at[pl.ds(s * ept, ept)])

        hc = chunks // 2
        npairs = hc // 2
        for half in range(2):
            pltpu.sync_copy(src_hbm.at[wid, pl.ds(half * hc, hc)], src_v)
            pltpu.sync_copy(dst_hbm.at[wid, pl.ds(half * hc, hc)], dst_v)
            if half == 0:
                plsc.subcore_barrier()
            def gather(j, b):
                pltpu.async_copy(h_hbm.at[src_v.at[j]], rows_v.at[b], gsem)

            def wait_gather(b):
                pltpu.make_async_copy(h_hbm.at[src_v.at[0]],
                                      rows_v.at[b], gsem).wait()

            gather(0, 0)

            def body(jj, carry):
                j0 = 2 * jj
                gather(j0 + 1, 1)
                wait_gather(0)
                pltpu.sync_copy(rows_v.at[0], agg_sh.at[dst_v.at[j0]], add=True)

                @pl.when(jj + 1 < npairs)
                def _():
                    gather(j0 + 2, 0)

                wait_gather(1)
                pltpu.sync_copy(rows_v.at[1], agg_sh.at[dst_v.at[j0 + 1]],
                                add=True)
                return carry

            lax.fori_loop(0, npairs, body, 0)
        plsc.subcore_barrier()
        pltpu.sync_copy(agg_sh.at[pl.ds(s * ept, ept)],
                        out_hbm.at[c, pl.ds(s * ept, ept)])

    return prop_kernel


def _dinv(d0_ref, d1_ref, n):
    d = d0_ref[...] + d1_ref[...] + 1.0
    return lax.rsqrt(d)[:n]


@functools.lru_cache(maxsize=None)
def _make_tc_in(n, d_in, d_hid):
    def body(x_ref, w_ref, d0_ref, d1_ref, out_ref):
        dinv = _dinv(d0_ref, d1_ref, n)
        h = jnp.dot(x_ref[...], w_ref[...], preferred_element_type=jnp.float32)
        out_ref[...] = h * dinv

    return pl.pallas_call(
        body, out_shape=jax.ShapeDtypeStruct((n, d_hid), jnp.float32))


@functools.lru_cache(maxsize=None)
def _make_tc_mid(n, n_pad, d, d_next):
    def body(p_ref, hp_ref, d0_ref, d1_ref, b_ref, g_ref, be_ref, w_ref, out_ref):
        dinv = _dinv(d0_ref, d1_ref, n)
        a = (p_ref[0, :n] + p_ref[1, :n] + hp_ref[...]) * dinv + b_ref[...]
        mu = jnp.mean(a, axis=0, keepdims=True)
        var = jnp.mean((a - mu) * (a - mu), axis=0, keepdims=True)
        y = (a - mu) * lax.rsqrt(var + 1e-5) * g_ref[...] + be_ref[...]
        y = jnp.maximum(y, 0.0)
        h = jnp.dot(y, w_ref[...], preferred_element_type=jnp.float32)
        out_ref[...] = h * dinv

    return pl.pallas_call(
        body, out_shape=jax.ShapeDtypeStruct((n, d_next), jnp.float32))


@functools.lru_cache(maxsize=None)
def _make_tc_mid_now(n, n_pad, d):
    def body(p_ref, hp_ref, d0_ref, d1_ref, b_ref, g_ref, be_ref, out_ref):
        dinv = _dinv(d0_ref, d1_ref, n)
        a = (p_ref[0, :n] + p_ref[1, :n] + hp_ref[...]) * dinv + b_ref[...]
        mu = jnp.mean(a, axis=0, keepdims=True)
        var = jnp.mean((a - mu) * (a - mu), axis=0, keepdims=True)
        y = (a - mu) * lax.rsqrt(var + 1e-5) * g_ref[...] + be_ref[...]
        y = jnp.maximum(y, 0.0)
        out_ref[...] = y * dinv

    return pl.pallas_call(
        body, out_shape=jax.ShapeDtypeStruct((n, d), jnp.float32))


@functools.lru_cache(maxsize=None)
def _make_tc_out(n, n_pad, d, d_out):
    def body(p_ref, hp_ref, d0_ref, d1_ref, w_ref, b_ref, out_ref):
        dinv = _dinv(d0_ref, d1_ref, n)
        agg = (p_ref[0, :n] + p_ref[1, :n] + hp_ref[...]) * dinv
        a = jnp.dot(agg, w_ref[...], preferred_element_type=jnp.float32)
        a = a + b_ref[...]
        m = jnp.max(a, axis=-1, keepdims=True)
        e = jnp.exp(a - m)
        lse = jnp.log(jnp.sum(e, axis=-1, keepdims=True))
        out_ref[...] = a - m - lse

    return pl.pallas_call(
        body, out_shape=jax.ShapeDtypeStruct((n, d_out), jnp.float32))


def kernel(x, edge_index, W1, b1, g1, be1, W2, b2, g2, be2, W3, b3):
    n, d_in = x.shape
    e = edge_index.shape[1]
    d_hid = W1.shape[1]
    d_out = W3.shape[1]

    n_pad = ((n + 256) // 256) * 256
    epw = NC * NS * K * 4
    e_pad = ((e + epw - 1) // epw) * epw
    chunks = e_pad // (NC * NS * K)
    pad = e_pad - e

    pad_idx = jnp.arange(pad, dtype=jnp.int32)
    src = jnp.concatenate([edge_index[0], pad_idx % n])
    dst = jnp.concatenate([edge_index[1], n + pad_idx % (n_pad - n)])
    src3 = src.reshape(NC * NS, chunks, K)
    dst3 = dst.reshape(NC * NS, chunks, K)

    zeros1 = jnp.zeros((n_pad,), jnp.float32)
    ones_k = jnp.ones((K,), jnp.float32)
    zeros_h = jnp.zeros((n_pad, d_hid), jnp.float32)

    deg_flat = _make_deg(n_pad, chunks)(dst3, ones_k, zeros1)
    deg0 = deg_flat[:n_pad].reshape(n_pad, 1)
    deg1 = deg_flat[n_pad:].reshape(n_pad, 1)

    h1p = _make_tc_in(n, d_in, d_hid)(x, W1, deg0, deg1)
    p1 = _make_prop(n_pad, d_hid, chunks)(h1p, src3, dst3, zeros_h)
    h2p = _make_tc_mid(n, n_pad, d_hid, d_hid)(p1, h1p, deg0, deg1, b1, g1, be1, W2)
    p2 = _make_prop(n_pad, d_hid, chunks)(h2p, src3, dst3, zeros_h)
    y2p = _make_tc_mid_now(n, n_pad, d_hid)(p2, h2p, deg0, deg1, b2, g2, be2)
    p3 = _make_prop(n_pad, d_hid, chunks)(y2p, src3, dst3, zeros_h)
    return _make_tc_out(n, n_pad, d_hid, d_out)(p3, y2p, deg0, deg1, W3, b3)
